# baseline (speedup 1.0000x reference)
import functools

import jax
import jax.numpy as jnp
from jax import lax
from jax.experimental import pallas as pl
from jax.experimental.pallas import tpu as pltpu
from jax.experimental.pallas import tpu_sc as plsc

VOCAB = 1000000
EMBED_DIM = 64
SEQ_LEN = 200
BATCH = 4096

NC = 2
NS = 16
LANES = 16
NW = NC * NS
BPW = BATCH // NW

BLKV = 8192
NBLK = -(-VOCAB // BLKV)


VHALF = VOCAB // 2
BLKH = 8192
NBLKH = -(-VHALF // BLKH)


def _matvec_body(t_ref, w2_ref, o_ref):
    o_ref[...] = jax.lax.dot_general(
        w2_ref[...], t_ref[...],
        dimension_numbers=(((1,), (1,)), ((), ())),
        preferred_element_type=jnp.float32,
    )


def _fold_table(tablec, w2):
    return pl.pallas_call(
        _matvec_body,
        grid=(NBLKH,),
        in_specs=[
            pl.BlockSpec(
                (BLKH, 2 * EMBED_DIM),
                lambda i: (jnp.minimum(i, NBLKH - 1), 0),
            ),
            pl.BlockSpec((2, 2 * EMBED_DIM), lambda i: (0, 0)),
        ],
        out_specs=pl.BlockSpec((2, BLKH), lambda i: (0, i)),
        out_shape=jax.ShapeDtypeStruct((2, VHALF), jnp.float32),
    )(tablec, w2)


_sc_mesh = plsc.VectorSubcoreMesh(core_axis_name="c", subcore_axis_name="s")


@functools.partial(
    pl.kernel,
    mesh=_sc_mesh,
    out_type=jax.ShapeDtypeStruct((BATCH,), jnp.float32),
    scratch_types=[
        pltpu.VMEM((SEQ_LEN, BPW), jnp.int32),
        pltpu.VMEM((SEQ_LEN, BPW), jnp.float32),
        pltpu.VMEM((BPW,), jnp.float32),
        pltpu.VMEM((LANES,), jnp.float32),
        pltpu.SemaphoreType.DMA,
    ],
)
def _sc_pool(tv_hbm, x_hbm, b_hbm, out_hbm, idx_v, g_v, o_v, b_v, sem):
    wid = lax.axis_index("s") * NC + lax.axis_index("c")
    base = wid * BPW

    pltpu.sync_copy(x_hbm.at[:, pl.ds(base, BPW)], idx_v)
    pltpu.sync_copy(b_hbm, b_v)

    def fire(l, c):
        pltpu.async_copy(tv_hbm.at[idx_v.at[l]], g_v.at[l], sem)
        return c

    lax.fori_loop(0, SEQ_LEN, fire, 0)

    def drain(l, c):
        pltpu.make_async_copy(tv_hbm.at[idx_v.at[l]], g_v.at[l], sem).wait()
        return c

    lax.fori_loop(0, SEQ_LEN, drain, 0)

    def body(l, accs):
        return tuple(
            a + g_v[l, pl.ds(j * LANES, LANES)] for j, a in enumerate(accs)
        )

    zero = jnp.zeros((LANES,), jnp.float32)
    accs = lax.fori_loop(0, SEQ_LEN, body, tuple(zero for _ in range(BPW // LANES)))

    inv_l = jnp.float32(1.0 / SEQ_LEN)
    bias = b_v[...]
    one = jnp.float32(1.0)
    for j, a in enumerate(accs):
        z = a * inv_l + bias
        o_v[pl.ds(j * LANES, LANES)] = one / (one + jnp.exp(-z))

    pltpu.sync_copy(o_v, out_hbm.at[pl.ds(base, BPW)])


def kernel(x, table, fc_w, fc_b):
    x32 = x.astype(jnp.int32)
    tablec = table.reshape(VHALF, 2 * EMBED_DIM)
    w2 = jnp.zeros((2, 2 * EMBED_DIM), jnp.float32)
    w2 = w2.at[0, :EMBED_DIM].set(fc_w[0])
    w2 = w2.at[1, EMBED_DIM:].set(fc_w[0])
    tv2 = _fold_table(tablec, w2)
    tv = tv2.T.reshape(VOCAB)
    bvec = jnp.broadcast_to(fc_b.astype(jnp.float32), (LANES,))
    return _sc_pool(tv, x32, bvec)


# speedup vs baseline: 1.8991x; 1.8991x over previous
import functools

import jax
import jax.numpy as jnp
from jax import lax
from jax.experimental import pallas as pl
from jax.experimental.pallas import tpu as pltpu
from jax.experimental.pallas import tpu_sc as plsc

VOCAB = 1000000
EMBED_DIM = 64
SEQ_LEN = 200
BATCH = 4096

NC = 2
NS = 16
LANES = 16
NW = NC * NS
BPW = BATCH // NW

BLKV = 8192
NBLK = -(-VOCAB // BLKV)


NSTREAM = 4
NBLK4 = -(-NBLK // NSTREAM)


def _matvec_body(t0, t1, t2, t3, w_ref, o_ref):
    w = w_ref[...]
    for k, t in enumerate((t0, t1, t2, t3)):
        o2 = jax.lax.dot_general(
            w, t[...],
            dimension_numbers=(((1,), (1,)), ((), ())),
            preferred_element_type=jnp.float32,
        )
        o_ref[pl.ds(k * BLKV, BLKV)] = o2[0]


def _fold_table(table, fc_w):
    def tspec(k):
        return pl.BlockSpec(
            (BLKV, EMBED_DIM),
            lambda i, k=k: (jnp.minimum(NSTREAM * i + k, NBLK - 1), 0),
        )

    return pl.pallas_call(
        _matvec_body,
        grid=(NBLK4,),
        in_specs=[tspec(0), tspec(1), tspec(2), tspec(3),
                  pl.BlockSpec((1, EMBED_DIM), lambda i: (0, 0))],
        out_specs=pl.BlockSpec((NSTREAM * BLKV,), lambda i: (i,)),
        out_shape=jax.ShapeDtypeStruct((VOCAB,), jnp.float32),
    )(table, table, table, table, fc_w)


_sc_mesh = plsc.VectorSubcoreMesh(core_axis_name="c", subcore_axis_name="s")


@functools.partial(
    pl.kernel,
    mesh=_sc_mesh,
    out_type=jax.ShapeDtypeStruct((BATCH,), jnp.float32),
    scratch_types=[
        pltpu.VMEM((SEQ_LEN, BPW), jnp.int32),
        pltpu.VMEM((SEQ_LEN, BPW), jnp.float32),
        pltpu.VMEM((BPW,), jnp.float32),
        pltpu.VMEM((LANES,), jnp.float32),
        pltpu.SemaphoreType.DMA,
    ],
)
def _sc_pool(tv_hbm, x_hbm, b_hbm, out_hbm, idx_v, g_v, o_v, b_v, sem):
    wid = lax.axis_index("s") * NC + lax.axis_index("c")
    base = wid * BPW

    pltpu.sync_copy(x_hbm.at[:, pl.ds(base, BPW)], idx_v)
    pltpu.sync_copy(b_hbm, b_v)

    def fire(l, c):
        pltpu.async_copy(tv_hbm.at[idx_v.at[l]], g_v.at[l], sem)
        return c

    lax.fori_loop(0, SEQ_LEN, fire, 0)

    def drain(l, c):
        pltpu.make_async_copy(tv_hbm.at[idx_v.at[l]], g_v.at[l], sem).wait()
        return c

    lax.fori_loop(0, SEQ_LEN, drain, 0)

    def body(l, accs):
        return tuple(
            a + g_v[l, pl.ds(j * LANES, LANES)] for j, a in enumerate(accs)
        )

    zero = jnp.zeros((LANES,), jnp.float32)
    accs = lax.fori_loop(0, SEQ_LEN, body, tuple(zero for _ in range(BPW // LANES)))

    inv_l = jnp.float32(1.0 / SEQ_LEN)
    bias = b_v[...]
    one = jnp.float32(1.0)
    for j, a in enumerate(accs):
        z = a * inv_l + bias
        o_v[pl.ds(j * LANES, LANES)] = one / (one + jnp.exp(-z))

    pltpu.sync_copy(o_v, out_hbm.at[pl.ds(base, BPW)])


def kernel(x, table, fc_w, fc_b):
    x32 = x.astype(jnp.int32)
    tv = _fold_table(table, fc_w)
    bvec = jnp.broadcast_to(fc_b.astype(jnp.float32), (LANES,))
    return _sc_pool(tv, x32, bvec)
